# baseline (speedup 1.0000x reference)
import functools

import jax
import jax.numpy as jnp
from jax.experimental import pallas as pl
from jax.experimental.pallas import tpu as pltpu

_K = 32
_R = 256
_C = 2048
_NEG = float("-inf")


def _normalize_body(x_ref, simf_ref, simft_ref):
    f = x_ref[:, :]
    ss = jnp.sum(f * f, axis=1, keepdims=True)
    nrm = jnp.sqrt(ss)
    s = (f / (nrm + 1e-10)).astype(jnp.bfloat16)
    simf_ref[:, :] = s
    simft_ref[:, :] = s.T


def _topk_body(n, n_col_blocks, simf_ref, simft_ref, out_ref,
               vals_ref, idx_ref, s_ref):
    r = pl.program_id(0)
    c = pl.program_id(1)

    @pl.when(c == 0)
    def _init():
        vals_ref[:, :] = jnp.full((_R, _K), _NEG, dtype=jnp.float32)
        idx_ref[:, :] = jnp.zeros((_R, _K), dtype=jnp.float32)

    a = simf_ref[:, :]
    b = simft_ref[:, pl.ds(pl.multiple_of(c * _C, _C), _C)]
    sim = jax.lax.dot_general(
        a, b, (((1,), (0,)), ((), ())),
        preferred_element_type=jnp.float32,
    )

    row_ids = r * _R + jax.lax.broadcasted_iota(jnp.int32, (_R, _C), 0)
    col_ids = c * _C + jax.lax.broadcasted_iota(jnp.int32, (_R, _C), 1)
    sim = jnp.where(row_ids == col_ids, 0.0, sim)
    sim = jnp.where(col_ids >= n, _NEG, sim)
    colf = -col_ids.astype(jnp.float32)

    cv = vals_ref[:, :]
    ci = idx_ref[:, :]

    cmin = jnp.min(cv, axis=1, keepdims=True)
    smask = sim > cmin
    s_ref[:, :] = jnp.where(smask, sim, _NEG)
    cnt = jnp.sum(smask.astype(jnp.float32), axis=1, keepdims=True)
    t = jnp.minimum(jnp.max(cnt), float(_K)).astype(jnp.int32)

    iota_k = jax.lax.broadcasted_iota(jnp.int32, (_R, _K), 1)

    def _extract(k, carry):
        tv, ti_t = carry
        s = s_ref[:, :]
        m = jnp.max(s, axis=1, keepdims=True)
        sel = jnp.max(jnp.where(s == m, colf, _NEG), axis=1, keepdims=True)
        s_ref[:, :] = jnp.where(colf == sel, _NEG, s)
        lane = _K - 1 - k
        tv = jnp.where(iota_k == lane, m, tv)
        ti_t = jnp.where(iota_k == lane, sel, ti_t)
        return tv, ti_t

    tv, ti_t = jax.lax.fori_loop(
        0, t, _extract,
        (jnp.full((_R, _K), _NEG, dtype=jnp.float32),
         jnp.zeros((_R, _K), dtype=jnp.float32)),
    )

    allv = jnp.concatenate([cv, tv], axis=1)
    alli = jnp.concatenate([ci, ti_t], axis=1)
    for k in range(_K):
        m = jnp.max(allv, axis=1, keepdims=True)
        sel = jnp.max(jnp.where(allv == m, alli, _NEG), axis=1, keepdims=True)
        vals_ref[:, k:k + 1] = m
        idx_ref[:, k:k + 1] = sel
        allv = jnp.where((allv == m) & (alli == sel), _NEG, allv)

    @pl.when(c == n_col_blocks - 1)
    def _emit():
        out_ref[:, :] = (-idx_ref[:, :]).astype(jnp.int32)


def _build_topk(n, np_, d):
    n_row_blocks = np_ // _R
    n_col_blocks = np_ // _C
    body = functools.partial(_topk_body, n, n_col_blocks)
    return pl.pallas_call(
        body,
        grid=(n_row_blocks, n_col_blocks),
        in_specs=[
            pl.BlockSpec((_R, d), lambda r, c: (r, 0)),
            pl.BlockSpec((d, np_), lambda r, c: (0, 0)),
        ],
        out_specs=pl.BlockSpec((_R, _K), lambda r, c: (r, 0)),
        out_shape=jax.ShapeDtypeStruct((np_, _K), jnp.int32),
        scratch_shapes=[
            pltpu.VMEM((_R, _K), jnp.float32),
            pltpu.VMEM((_R, _K), jnp.float32),
            pltpu.VMEM((_R, _C), jnp.float32),
        ],
        compiler_params=pltpu.CompilerParams(
            dimension_semantics=("arbitrary", "arbitrary"),
        ),
    )


def kernel(feature):
    n, d = feature.shape
    np_ = ((n + _C - 1) // _C) * _C
    fpad = jnp.pad(feature, ((0, np_ - n), (0, 0)))

    simf, simft = pl.pallas_call(
        _normalize_body,
        out_shape=(
            jax.ShapeDtypeStruct((np_, d), jnp.bfloat16),
            jax.ShapeDtypeStruct((d, np_), jnp.bfloat16),
        ),
    )(fpad)

    idx = _build_topk(n, np_, d)(simf, simft)
    idx = idx[:n]
    rows = jnp.repeat(jnp.arange(n), _K)
    edge = jnp.stack([rows, idx.reshape(-1)], axis=0).astype(jnp.int64)
    return edge


# speedup vs baseline: 1.4968x; 1.4968x over previous
import functools

import jax
import jax.numpy as jnp
from jax.experimental import pallas as pl
from jax.experimental.pallas import tpu as pltpu

_K = 32
_R = 256
_C = 2048
_NEG = float("-inf")


def _normalize_body(x_ref, simf_ref, simft_ref):
    f = x_ref[:, :]
    ss = jnp.sum(f * f, axis=1, keepdims=True)
    nrm = jnp.sqrt(ss)
    s = (f / (nrm + 1e-10)).astype(jnp.bfloat16)
    simf_ref[:, :] = s
    simft_ref[:, :] = s.T


def _topk_body(n, n_col_blocks, simf_ref, simft_ref, out_ref, vals_ref, idx_ref):
    r = pl.program_id(0)
    c = pl.program_id(1)

    @pl.when(c == 0)
    def _init():
        vals_ref[:, :] = jnp.full((_R, _K), _NEG, dtype=jnp.float32)
        idx_ref[:, :] = jnp.zeros((_R, _K), dtype=jnp.float32)

    a = simf_ref[:, :]
    b = simft_ref[:, pl.ds(pl.multiple_of(c * _C, _C), _C)]
    sim = jax.lax.dot_general(
        a, b, (((1,), (0,)), ((), ())),
        preferred_element_type=jnp.float32,
    )

    row_ids = r * _R + jax.lax.broadcasted_iota(jnp.int32, (_R, _C), 0)
    col_ids = c * _C + jax.lax.broadcasted_iota(jnp.int32, (_R, _C), 1)
    sim = jnp.where(row_ids == col_ids, 0.0, sim)
    sim = jnp.where(col_ids >= n, _NEG, sim)
    colf = -col_ids.astype(jnp.float32)

    cv = vals_ref[:, :]
    ci = idx_ref[:, :]

    for k in range(_K):
        mt = jnp.max(sim, axis=1, keepdims=True)
        mc = jnp.max(cv, axis=1, keepdims=True)
        m = jnp.maximum(mt, mc)
        in_carry = mc >= mt
        icand_c = jnp.max(jnp.where(cv == m, ci, _NEG), axis=1, keepdims=True)
        icand_t = jnp.max(jnp.where(sim == m, colf, _NEG), axis=1, keepdims=True)
        sel = jnp.where(in_carry, icand_c, icand_t)
        vals_ref[:, k:k + 1] = m
        idx_ref[:, k:k + 1] = sel
        cv = jnp.where(ci == sel, _NEG, cv)
        sim = jnp.where(colf == sel, _NEG, sim)

    @pl.when(c == n_col_blocks - 1)
    def _emit():
        out_ref[:, :] = (-idx_ref[:, :]).astype(jnp.int32)


def _build_topk(n, np_, d):
    n_row_blocks = np_ // _R
    n_col_blocks = np_ // _C
    body = functools.partial(_topk_body, n, n_col_blocks)
    return pl.pallas_call(
        body,
        grid=(n_row_blocks, n_col_blocks),
        in_specs=[
            pl.BlockSpec((_R, d), lambda r, c: (r, 0)),
            pl.BlockSpec((d, np_), lambda r, c: (0, 0)),
        ],
        out_specs=pl.BlockSpec((_R, _K), lambda r, c: (r, 0)),
        out_shape=jax.ShapeDtypeStruct((np_, _K), jnp.int32),
        scratch_shapes=[
            pltpu.VMEM((_R, _K), jnp.float32),
            pltpu.VMEM((_R, _K), jnp.float32),
        ],
        compiler_params=pltpu.CompilerParams(
            dimension_semantics=("arbitrary", "arbitrary"),
        ),
    )


def kernel(feature):
    n, d = feature.shape
    np_ = ((n + _C - 1) // _C) * _C
    fpad = jnp.pad(feature, ((0, np_ - n), (0, 0)))

    simf, simft = pl.pallas_call(
        _normalize_body,
        out_shape=(
            jax.ShapeDtypeStruct((np_, d), jnp.bfloat16),
            jax.ShapeDtypeStruct((d, np_), jnp.bfloat16),
        ),
    )(fpad)

    idx = _build_topk(n, np_, d)(simf, simft)
    idx = idx[:n]
    rows = jnp.repeat(jnp.arange(n), _K)
    edge = jnp.stack([rows, idx.reshape(-1)], axis=0).astype(jnp.int64)
    return edge
